# baseline (speedup 1.0000x reference)
import jax
import jax.numpy as jnp
from jax.experimental import pallas as pl
from jax.experimental.pallas import tpu as pltpu

_B = 16
_CHUNK = 16384


def _body(lower_ref, upper_ref, invc_ref, flat_ref,
          w1_ref, w2_ref, w3_ref, w4_ref, b4_ref, w5_ref, b5_ref,
          out_ref, h1_ref, h2_ref):
    i = pl.program_id(0)

    @pl.when(i == 0)
    def _():
        ones_col = (jax.lax.broadcasted_iota(jnp.int32, (_CHUNK, 8), 1)
                    == 0).astype(jnp.bfloat16)
        h1_ref[:, 64:72] = ones_col
        h2_ref[:, 128:136] = ones_col

    x = flat_ref[...]
    h1_ref[:, :64] = jnp.maximum(
        jnp.dot(x, w1_ref[...], preferred_element_type=jnp.float32),
        0.0).astype(jnp.bfloat16)
    h2_ref[:, :128] = jnp.maximum(
        jnp.dot(h1_ref[...], w2_ref[...],
                preferred_element_type=jnp.float32), 0.0).astype(jnp.bfloat16)
    h = jnp.maximum(
        jnp.dot(h2_ref[...], w3_ref[...],
                preferred_element_type=jnp.float32), 0.0).astype(jnp.bfloat16)

    row = jax.lax.broadcasted_iota(jnp.int32, (_B, _CHUNK), 1) + i * _CHUNK
    member = (row >= lower_ref[...]) & (row < upper_ref[...])
    onehot = member.astype(jnp.bfloat16)
    sums = jnp.dot(onehot, h, preferred_element_type=jnp.float32)

    pooled = (sums * invc_ref[...]).astype(jnp.bfloat16)
    z = jnp.maximum(
        jnp.dot(pooled, w4_ref[...], preferred_element_type=jnp.float32)
        + b4_ref[...], 0.0).astype(jnp.bfloat16)
    o = jnp.dot(z, w5_ref[...], preferred_element_type=jnp.float32) \
        + b5_ref[...]
    out_ref[...] = jax.nn.sigmoid(o)


def kernel(flat, cu_seqlens, W1, b1, W2, b2, W3, b3, W4, b4, W5, b5):
    total, d_in = flat.shape
    n_chunks = total // _CHUNK

    bf = jnp.bfloat16
    ones = jnp.ones((total, 1), jnp.float32)
    flat_a = jnp.concatenate(
        [flat, ones, jnp.zeros((total, 4), jnp.float32)], axis=1).astype(bf)
    w1_a = jnp.concatenate(
        [W1, b1[None, :], jnp.zeros((4, 64), jnp.float32)], axis=0).astype(bf)
    w2_a = jnp.concatenate(
        [W2, b2[None, :], jnp.zeros((7, 128), jnp.float32)], axis=0).astype(bf)
    w3_a = jnp.concatenate(
        [W3, b3[None, :], jnp.zeros((7, 256), jnp.float32)], axis=0).astype(bf)

    lower = cu_seqlens[:-1].reshape(_B, 1)
    upper = cu_seqlens[1:].reshape(_B, 1)
    counts = (upper - lower).astype(jnp.float32)
    invc = 1.0 / jnp.maximum(counts, 1.0)

    full = lambda shape: pl.BlockSpec(shape, lambda i: (0, 0))
    in_specs = [
        full((_B, 1)),
        full((_B, 1)),
        full((_B, 1)),
        pl.BlockSpec((_CHUNK, 8), lambda i: (i, 0)),
        full(w1_a.shape), full(w2_a.shape), full(w3_a.shape),
        full(W4.shape), full((1, 512)), full(W5.shape), full((1, 2500)),
    ]
    return pl.pallas_call(
        _body,
        grid=(n_chunks,),
        in_specs=in_specs,
        out_specs=full((_B, 2500)),
        out_shape=jax.ShapeDtypeStruct((_B, 2500), jnp.float32),
        scratch_shapes=[
            pltpu.VMEM((_CHUNK, 72), bf),
            pltpu.VMEM((_CHUNK, 136), bf),
        ],
        compiler_params=pltpu.CompilerParams(
            dimension_semantics=("arbitrary",)),
    )(lower, upper, invc, flat_a, w1_a, w2_a, w3_a,
      W4.astype(bf), b4.reshape(1, 512), W5.astype(bf), b5.reshape(1, 2500))


# speedup vs baseline: 1.1787x; 1.1787x over previous
import jax
import jax.numpy as jnp
from jax.experimental import pallas as pl
from jax.experimental.pallas import tpu as pltpu

_B = 16
_CHUNK = 4096


def _body(lower_ref, upper_ref, invc_ref, flat_ref,
          w1_ref, w2_ref, w3_ref, w4_ref, b4_ref, w5_ref, b5_ref,
          out_ref, acc_ref, h1_ref, h2_ref):
    i = pl.program_id(0)

    x = flat_ref[...]
    h1_ref[...] = jnp.maximum(
        jnp.dot(x, w1_ref[...], preferred_element_type=jnp.float32), 0.0)
    h2_ref[...] = jnp.maximum(
        jnp.dot(h1_ref[...], w2_ref[...],
                preferred_element_type=jnp.float32), 0.0)
    h = jnp.maximum(
        jnp.dot(h2_ref[...], w3_ref[...],
                preferred_element_type=jnp.float32), 0.0)

    row = jax.lax.broadcasted_iota(jnp.int32, (_B, _CHUNK), 1) + i * _CHUNK
    member = (row >= lower_ref[...]) & (row < upper_ref[...])
    onehot = member.astype(jnp.float32) * invc_ref[...]
    part = jnp.dot(onehot, h, preferred_element_type=jnp.float32)

    @pl.when(i == 0)
    def _():
        acc_ref[...] = part

    @pl.when(i > 0)
    def _():
        acc_ref[...] += part

    @pl.when(i == pl.num_programs(0) - 1)
    def _():
        pooled = acc_ref[...]
        z = jnp.maximum(
            jnp.dot(pooled, w4_ref[...], preferred_element_type=jnp.float32)
            + b4_ref[...], 0.0)
        o = jnp.dot(z, w5_ref[...], preferred_element_type=jnp.float32) \
            + b5_ref[...]
        out_ref[...] = jax.nn.sigmoid(o)


def kernel(flat, cu_seqlens, W1, b1, W2, b2, W3, b3, W4, b4, W5, b5):
    total, d_in = flat.shape
    n_chunks = total // _CHUNK

    ones = jnp.ones((total, 1), jnp.float32)
    flat_a = jnp.concatenate(
        [flat, ones, jnp.zeros((total, 4), jnp.float32)], axis=1)

    def aug(W, b, k_pad, fwd_row):
        k, n = W.shape
        Wb = jnp.concatenate(
            [W, b[None, :], jnp.zeros((k_pad, n), jnp.float32)], axis=0)
        unit = jnp.zeros((k + 1 + k_pad, 8), jnp.float32)
        unit = unit.at[fwd_row, 0].set(1.0)
        return jnp.concatenate([Wb, unit], axis=1)

    w1_a = aug(W1, b1, 4, 3)
    w2_a = aug(W2, b2, 7, 64)
    w3_a = jnp.concatenate(
        [W3, b3[None, :], jnp.zeros((135 - W3.shape[0], 256), jnp.float32)],
        axis=0)

    lower = cu_seqlens[:-1].reshape(_B, 1)
    upper = cu_seqlens[1:].reshape(_B, 1)
    counts = (upper - lower).astype(jnp.float32)
    invc = 1.0 / jnp.maximum(counts, 1.0)

    full = lambda shape: pl.BlockSpec(shape, lambda i: (0, 0))
    in_specs = [
        full((_B, 1)),
        full((_B, 1)),
        full((_B, 1)),
        pl.BlockSpec((_CHUNK, 8), lambda i: (i, 0)),
        full(w1_a.shape), full(w2_a.shape), full(w3_a.shape),
        full(W4.shape), full((1, 512)), full(W5.shape), full((1, 2500)),
    ]
    return pl.pallas_call(
        _body,
        grid=(n_chunks,),
        in_specs=in_specs,
        out_specs=full((_B, 2500)),
        out_shape=jax.ShapeDtypeStruct((_B, 2500), jnp.float32),
        scratch_shapes=[
            pltpu.VMEM((_B, 256), jnp.float32),
            pltpu.VMEM((_CHUNK, 72), jnp.float32),
            pltpu.VMEM((_CHUNK, 136), jnp.float32),
        ],
        compiler_params=pltpu.CompilerParams(
            dimension_semantics=("arbitrary",)),
    )(lower, upper, invc, flat_a, w1_a, w2_a, w3_a,
      W4, b4.reshape(1, 512), W5, b5.reshape(1, 2500))


# speedup vs baseline: 1.3738x; 1.1655x over previous
import jax
import jax.numpy as jnp
from jax.experimental import pallas as pl
from jax.experimental.pallas import tpu as pltpu

_B = 16
_CHUNK = 4096


def _body(cu_ref, flat_ref, w1_ref, b1_ref, w2_ref, b2_ref, w3_ref, b3_ref,
          w4_ref, b4_ref, w5_ref, b5_ref, out_ref,
          w1s_ref, b1s_ref, w2s_ref, w3s_ref, acc_ref, h1_ref, h2_ref):
    i = pl.program_id(0)

    @pl.when(i == 0)
    def _():
        w1s_ref[...] = jnp.zeros_like(w1s_ref)
        w1s_ref[0:3, 0:64] = w1_ref[...]
        b1s_ref[...] = (jax.lax.broadcasted_iota(jnp.int32, (1, 72), 1)
                        == 64).astype(jnp.float32)
        b1s_ref[0:1, 0:64] = b1_ref[...]
        w2s_ref[...] = jnp.zeros_like(w2s_ref)
        w2s_ref[0:64, 0:128] = w2_ref[...]
        w2s_ref[64:65, 0:128] = b2_ref[...]
        w2s_ref[64:65, 128:136] = (
            jax.lax.broadcasted_iota(jnp.int32, (1, 8), 1)
            == 0).astype(jnp.float32)
        w3s_ref[...] = jnp.zeros_like(w3s_ref)
        w3s_ref[0:128, :] = w3_ref[...]
        w3s_ref[128:129, :] = b3_ref[...]

    x = flat_ref[...]
    h1_ref[...] = jnp.maximum(
        jnp.dot(x, w1s_ref[0:3, :], preferred_element_type=jnp.float32)
        + b1s_ref[...], 0.0)
    h2_ref[...] = jnp.maximum(
        jnp.dot(h1_ref[...], w2s_ref[...],
                preferred_element_type=jnp.float32), 0.0)
    h = jnp.maximum(
        jnp.dot(h2_ref[...], w3s_ref[...],
                preferred_element_type=jnp.float32), 0.0)

    cu = cu_ref[...]
    lo = cu[:16]
    hi = cu[1:17]
    invc = 1.0 / jnp.maximum(hi - lo, 1).astype(jnp.float32)
    row = jax.lax.broadcasted_iota(jnp.int32, (_CHUNK, _B), 0) + i * _CHUNK
    member = (row >= lo) & (row < hi)
    onehot = member.astype(jnp.float32) * invc
    part = jax.lax.dot_general(
        onehot, h, (((0,), (0,)), ((), ())),
        preferred_element_type=jnp.float32)

    @pl.when(i == 0)
    def _():
        acc_ref[...] = part

    @pl.when(i > 0)
    def _():
        acc_ref[...] += part

    @pl.when(i == pl.num_programs(0) - 1)
    def _():
        pooled = acc_ref[...]
        z = jnp.maximum(
            jnp.dot(pooled, w4_ref[...], preferred_element_type=jnp.float32)
            + b4_ref[...], 0.0)
        o = jnp.dot(z, w5_ref[...], preferred_element_type=jnp.float32) \
            + b5_ref[...]
        out_ref[...] = jax.nn.sigmoid(o)


def kernel(flat, cu_seqlens, W1, b1, W2, b2, W3, b3, W4, b4, W5, b5):
    total, d_in = flat.shape
    n_chunks = total // _CHUNK

    full = lambda shape: pl.BlockSpec(shape, lambda i: tuple(0 for _ in shape))
    in_specs = [
        pl.BlockSpec((17,), lambda i: (0,)),
        pl.BlockSpec((_CHUNK, 3), lambda i: (i, 0)),
        full(W1.shape), full((1, 64)),
        full(W2.shape), full((1, 128)),
        full(W3.shape), full((1, 256)),
        full(W4.shape), full((1, 512)),
        full(W5.shape), full((1, 2500)),
    ]
    return pl.pallas_call(
        _body,
        grid=(n_chunks,),
        in_specs=in_specs,
        out_specs=full((_B, 2500)),
        out_shape=jax.ShapeDtypeStruct((_B, 2500), jnp.float32),
        scratch_shapes=[
            pltpu.VMEM((8, 72), jnp.float32),
            pltpu.VMEM((1, 72), jnp.float32),
            pltpu.VMEM((72, 136), jnp.float32),
            pltpu.VMEM((136, 256), jnp.float32),
            pltpu.VMEM((_B, 256), jnp.float32),
            pltpu.VMEM((_CHUNK, 72), jnp.float32),
            pltpu.VMEM((_CHUNK, 136), jnp.float32),
        ],
        compiler_params=pltpu.CompilerParams(
            dimension_semantics=("arbitrary",)),
    )(cu_seqlens, flat,
      W1, b1.reshape(1, 64), W2, b2.reshape(1, 128), W3, b3.reshape(1, 256),
      W4, b4.reshape(1, 512), W5, b5.reshape(1, 2500))


# speedup vs baseline: 1.4471x; 1.0533x over previous
import jax
import jax.numpy as jnp
from jax.experimental import pallas as pl
from jax.experimental.pallas import tpu as pltpu

_B = 16
_CHUNK = 4096


def _body(cu_ref, flat_ref, w1_ref, b1_ref, w2_ref, b2_ref, w3_ref, b3_ref,
          w4_ref, b4_ref, w5_ref, b5_ref, out_ref,
          w1s_ref, b1s_ref, w2s_ref, w3s_ref, acc_ref, h1_ref, h2_ref):
    i = pl.program_id(0)

    @pl.when(i == 0)
    def _():
        w1s_ref[...] = jnp.zeros_like(w1s_ref)
        w1s_ref[0:3, 0:64] = w1_ref[...]
        b1s_ref[...] = (jax.lax.broadcasted_iota(jnp.int32, (1, 72), 1)
                        == 64).astype(jnp.float32)
        b1s_ref[0:1, 0:64] = b1_ref[...].reshape(1, 64)
        w2s_ref[...] = jnp.zeros_like(w2s_ref)
        w2s_ref[0:64, 0:128] = w2_ref[...]
        w2s_ref[64:65, 0:128] = b2_ref[...].reshape(1, 128)
        w2s_ref[64:65, 128:136] = (
            jax.lax.broadcasted_iota(jnp.int32, (1, 8), 1)
            == 0).astype(jnp.float32)
        w3s_ref[...] = jnp.zeros_like(w3s_ref)
        w3s_ref[0:128, :] = w3_ref[...]
        w3s_ref[128:129, :] = b3_ref[...].reshape(1, 256)

    x = flat_ref[...]
    h1_ref[...] = jnp.maximum(
        jnp.dot(x, w1s_ref[0:3, :], preferred_element_type=jnp.float32)
        + b1s_ref[...], 0.0)
    h2_ref[...] = jnp.maximum(
        jnp.dot(h1_ref[...], w2s_ref[...],
                preferred_element_type=jnp.float32), 0.0)
    h = jnp.maximum(
        jnp.dot(h2_ref[...], w3s_ref[...],
                preferred_element_type=jnp.float32), 0.0)

    cu = cu_ref[...]
    lo = cu[:16]
    hi = cu[1:17]
    invc = 1.0 / jnp.maximum(hi - lo, 1).astype(jnp.float32)
    row = jax.lax.broadcasted_iota(jnp.int32, (_CHUNK, _B), 0) + i * _CHUNK
    member = (row >= lo) & (row < hi)
    onehot = member.astype(jnp.float32) * invc
    part = jax.lax.dot_general(
        onehot, h, (((0,), (0,)), ((), ())),
        preferred_element_type=jnp.float32)

    @pl.when(i == 0)
    def _():
        acc_ref[...] = part

    @pl.when(i > 0)
    def _():
        acc_ref[...] += part

    @pl.when(i == pl.num_programs(0) - 1)
    def _():
        pooled = acc_ref[...]
        z = jnp.maximum(
            jnp.dot(pooled, w4_ref[...], preferred_element_type=jnp.float32)
            + b4_ref[...], 0.0)
        o = jnp.dot(z, w5_ref[...], preferred_element_type=jnp.float32) \
            + b5_ref[...]
        out_ref[...] = jax.nn.sigmoid(o)


def kernel(flat, cu_seqlens, W1, b1, W2, b2, W3, b3, W4, b4, W5, b5):
    total, d_in = flat.shape
    n_chunks = total // _CHUNK

    full = lambda shape: pl.BlockSpec(shape, lambda i: tuple(0 for _ in shape))
    in_specs = [
        pl.BlockSpec((17,), lambda i: (0,)),
        pl.BlockSpec((_CHUNK, 3), lambda i: (i, 0)),
        full(W1.shape), full((64,)),
        full(W2.shape), full((128,)),
        full(W3.shape), full((256,)),
        full(W4.shape), full((512,)),
        full(W5.shape), full((2500,)),
    ]
    return pl.pallas_call(
        _body,
        grid=(n_chunks,),
        in_specs=in_specs,
        out_specs=full((_B, 2500)),
        out_shape=jax.ShapeDtypeStruct((_B, 2500), jnp.float32),
        scratch_shapes=[
            pltpu.VMEM((8, 72), jnp.float32),
            pltpu.VMEM((1, 72), jnp.float32),
            pltpu.VMEM((72, 136), jnp.float32),
            pltpu.VMEM((136, 256), jnp.float32),
            pltpu.VMEM((_B, 256), jnp.float32),
            pltpu.VMEM((_CHUNK, 72), jnp.float32),
            pltpu.VMEM((_CHUNK, 136), jnp.float32),
        ],
        compiler_params=pltpu.CompilerParams(
            dimension_semantics=("arbitrary",)),
    )(cu_seqlens, flat, W1, b1, W2, b2, W3, b3, W4, b4, W5, b5)


# speedup vs baseline: 2.7439x; 1.8962x over previous
import jax
import jax.numpy as jnp
from jax.experimental import pallas as pl
from jax.experimental.pallas import tpu as pltpu

_B = 16
_CHUNK = 4096


def _body(cu_ref, flat_ref, w1_ref, b1_ref, w2_ref, b2_ref, w3_ref, b3_ref,
          w4_ref, b4_ref, w5t_ref, b5_ref, out_ref,
          w1s_ref, b1s_ref, w2s_ref, w3s_ref, acc_ref, h1_ref, h2_ref):
    i = pl.program_id(0)

    @pl.when(i == 0)
    def _():
        w1s_ref[...] = jnp.zeros_like(w1s_ref)
        w1s_ref[0:3, 0:64] = w1_ref[...]
        b1s_ref[...] = (jax.lax.broadcasted_iota(jnp.int32, (1, 72), 1)
                        == 64).astype(jnp.float32)
        b1s_ref[0:1, 0:64] = b1_ref[...].reshape(1, 64)
        w2s_ref[...] = jnp.zeros_like(w2s_ref)
        w2s_ref[0:64, 0:128] = w2_ref[...]
        w2s_ref[64:65, 0:128] = b2_ref[...].reshape(1, 128)
        w2s_ref[64:65, 128:136] = (
            jax.lax.broadcasted_iota(jnp.int32, (1, 8), 1)
            == 0).astype(jnp.float32)
        w3s_ref[...] = jnp.zeros_like(w3s_ref)
        w3s_ref[0:128, :] = w3_ref[...]
        w3s_ref[128:129, :] = b3_ref[...].reshape(1, 256)

    xt = flat_ref[...]
    h1_ref[...] = jnp.maximum(
        jax.lax.dot_general(xt, w1s_ref[0:3, :], (((0,), (0,)), ((), ())),
                            preferred_element_type=jnp.float32)
        + b1s_ref[...], 0.0)
    h2_ref[...] = jnp.maximum(
        jnp.dot(h1_ref[...], w2s_ref[...],
                preferred_element_type=jnp.float32), 0.0)
    h = jnp.maximum(
        jnp.dot(h2_ref[...], w3s_ref[...],
                preferred_element_type=jnp.float32), 0.0)

    cu = cu_ref[...]
    lo = cu[:16]
    hi = cu[1:17]
    invc = 1.0 / jnp.maximum(hi - lo, 1).astype(jnp.float32)
    row = jax.lax.broadcasted_iota(jnp.int32, (_CHUNK, _B), 0) + i * _CHUNK
    member = (row >= lo) & (row < hi)
    onehot = member.astype(jnp.float32) * invc
    part = jax.lax.dot_general(
        onehot, h, (((0,), (0,)), ((), ())),
        preferred_element_type=jnp.float32)

    @pl.when(i == 0)
    def _():
        acc_ref[...] = part

    @pl.when(i > 0)
    def _():
        acc_ref[...] += part

    @pl.when(i == pl.num_programs(0) - 1)
    def _():
        pooled = acc_ref[...]
        z = jnp.maximum(
            jnp.dot(pooled, w4_ref[...], preferred_element_type=jnp.float32)
            + b4_ref[...], 0.0)
        o = jax.lax.dot_general(
            z, w5t_ref[...], (((1,), (1,)), ((), ())),
            preferred_element_type=jnp.float32) + b5_ref[...]
        out_ref[...] = jax.nn.sigmoid(o)


def kernel(flat, cu_seqlens, W1, b1, W2, b2, W3, b3, W4, b4, W5, b5):
    total, d_in = flat.shape
    n_chunks = total // _CHUNK

    full = lambda shape: pl.BlockSpec(shape, lambda i: tuple(0 for _ in shape))
    in_specs = [
        pl.BlockSpec((17,), lambda i: (0,)),
        pl.BlockSpec((3, _CHUNK), lambda i: (0, i)),
        full(W1.shape), full((64,)),
        full(W2.shape), full((128,)),
        full(W3.shape), full((256,)),
        full(W4.shape), full((512,)),
        full((2500, 512)), full((2500,)),
    ]
    return pl.pallas_call(
        _body,
        grid=(n_chunks,),
        in_specs=in_specs,
        out_specs=full((_B, 2500)),
        out_shape=jax.ShapeDtypeStruct((_B, 2500), jnp.float32),
        scratch_shapes=[
            pltpu.VMEM((8, 72), jnp.float32),
            pltpu.VMEM((1, 72), jnp.float32),
            pltpu.VMEM((72, 136), jnp.float32),
            pltpu.VMEM((136, 256), jnp.float32),
            pltpu.VMEM((_B, 256), jnp.float32),
            pltpu.VMEM((_CHUNK, 72), jnp.float32),
            pltpu.VMEM((_CHUNK, 136), jnp.float32),
        ],
        compiler_params=pltpu.CompilerParams(
            dimension_semantics=("arbitrary",)),
    )(cu_seqlens, flat.T, W1, b1, W2, b2, W3, b3, W4, b4, W5.T, b5)


# speedup vs baseline: 2.8390x; 1.0347x over previous
import jax
import jax.numpy as jnp
from jax.experimental import pallas as pl
from jax.experimental.pallas import tpu as pltpu

_B = 16
_CHUNK = 4096


def _body(cu_ref, flat_ref, w1_ref, b1_ref, w2_ref, b2_ref, w3_ref, b3_ref,
          w4_ref, b4_ref, w5t_ref, b5_ref, out_ref,
          w1s_ref, b1s_ref, w2s_ref, w3s_ref, acc_ref, h1_ref, h2_ref):
    i = pl.program_id(0)

    @pl.when(i == 0)
    def _():
        w1s_ref[...] = jnp.zeros_like(w1s_ref)
        w1s_ref[0:3, 0:64] = w1_ref[...]
        b1s_ref[...] = (jax.lax.broadcasted_iota(jnp.int32, (1, 72), 1)
                        == 64).astype(jnp.float32)
        b1s_ref[0:1, 0:64] = b1_ref[...].reshape(1, 64)
        w2s_ref[...] = jnp.zeros_like(w2s_ref)
        w2s_ref[0:64, 0:128] = w2_ref[...]
        w2s_ref[64:65, 0:128] = b2_ref[...].reshape(1, 128)
        w2s_ref[64:65, 128:136] = (
            jax.lax.broadcasted_iota(jnp.int32, (1, 8), 1)
            == 0).astype(jnp.float32)
        w3s_ref[...] = jnp.zeros_like(w3s_ref)
        w3s_ref[0:128, :] = w3_ref[...]
        w3s_ref[128:129, :] = b3_ref[...].reshape(1, 256)

    xt = flat_ref[...]
    h1_ref[...] = jnp.maximum(
        jax.lax.dot_general(xt, w1s_ref[0:3, :], (((0,), (0,)), ((), ())),
                            preferred_element_type=jnp.float32)
        + b1s_ref[...], 0.0)
    h2_ref[...] = jnp.maximum(
        jnp.dot(h1_ref[...], w2s_ref[...],
                preferred_element_type=jnp.float32), 0.0)
    h = jnp.maximum(
        jnp.dot(h2_ref[...], w3s_ref[...],
                preferred_element_type=jnp.float32), 0.0)

    cut = jnp.transpose(cu_ref[...].reshape(1, 17))
    lo = cut[:16]
    hi = cut[1:17]
    invc = 1.0 / jnp.maximum(hi - lo, 1).astype(jnp.float32)
    row = jax.lax.broadcasted_iota(jnp.int32, (_B, _CHUNK), 1) + i * _CHUNK
    member = (row >= lo) & (row < hi)
    onehot = member.astype(jnp.float32) * invc
    part = jnp.dot(onehot, h, preferred_element_type=jnp.float32)

    @pl.when(i == 0)
    def _():
        acc_ref[...] = part

    @pl.when(i > 0)
    def _():
        acc_ref[...] += part

    @pl.when(i == pl.num_programs(0) - 1)
    def _():
        pooled = acc_ref[...]
        z = jnp.maximum(
            jnp.dot(pooled, w4_ref[...], preferred_element_type=jnp.float32)
            + b4_ref[...], 0.0)
        o = jax.lax.dot_general(
            z, w5t_ref[...], (((1,), (1,)), ((), ())),
            preferred_element_type=jnp.float32) + b5_ref[...]
        out_ref[...] = jax.nn.sigmoid(o)


def kernel(flat, cu_seqlens, W1, b1, W2, b2, W3, b3, W4, b4, W5, b5):
    total, d_in = flat.shape
    n_chunks = total // _CHUNK

    full = lambda shape: pl.BlockSpec(shape, lambda i: tuple(0 for _ in shape))
    in_specs = [
        pl.BlockSpec((17,), lambda i: (0,)),
        pl.BlockSpec((3, _CHUNK), lambda i: (0, i)),
        full(W1.shape), full((64,)),
        full(W2.shape), full((128,)),
        full(W3.shape), full((256,)),
        full(W4.shape), full((512,)),
        full((2500, 512)), full((2500,)),
    ]
    return pl.pallas_call(
        _body,
        grid=(n_chunks,),
        in_specs=in_specs,
        out_specs=full((_B, 2500)),
        out_shape=jax.ShapeDtypeStruct((_B, 2500), jnp.float32),
        scratch_shapes=[
            pltpu.VMEM((8, 72), jnp.float32),
            pltpu.VMEM((1, 72), jnp.float32),
            pltpu.VMEM((72, 136), jnp.float32),
            pltpu.VMEM((136, 256), jnp.float32),
            pltpu.VMEM((_B, 256), jnp.float32),
            pltpu.VMEM((_CHUNK, 72), jnp.float32),
            pltpu.VMEM((_CHUNK, 136), jnp.float32),
        ],
        compiler_params=pltpu.CompilerParams(
            dimension_semantics=("arbitrary",)),
    )(cu_seqlens, flat.T, W1, b1, W2, b2, W3, b3, W4, b4, W5.T, b5)


# speedup vs baseline: 2.9895x; 1.0530x over previous
import jax
import jax.numpy as jnp
from jax.experimental import pallas as pl
from jax.experimental.pallas import tpu as pltpu

_B = 16
_CHUNK = 8192


def _body(cu_ref, flat_ref, w1_ref, b1_ref, w2_ref, b2_ref, w3_ref, b3_ref,
          w4_ref, b4_ref, w5t_ref, b5_ref, out_ref,
          w1s_ref, b1s_ref, w2s_ref, w3s_ref, acc_ref, h1_ref, h2_ref):
    i = pl.program_id(0)

    @pl.when(i == 0)
    def _():
        w1s_ref[...] = jnp.zeros_like(w1s_ref)
        w1s_ref[0:3, 0:64] = w1_ref[...]
        b1s_ref[...] = (jax.lax.broadcasted_iota(jnp.int32, (1, 72), 1)
                        == 64).astype(jnp.float32)
        b1s_ref[0:1, 0:64] = b1_ref[...].reshape(1, 64)
        w2s_ref[...] = jnp.zeros_like(w2s_ref)
        w2s_ref[0:64, 0:128] = w2_ref[...]
        w2s_ref[64:65, 0:128] = b2_ref[...].reshape(1, 128)
        w2s_ref[64:65, 128:136] = (
            jax.lax.broadcasted_iota(jnp.int32, (1, 8), 1)
            == 0).astype(jnp.float32)
        w3s_ref[...] = jnp.zeros_like(w3s_ref)
        w3s_ref[0:128, :] = w3_ref[...]
        w3s_ref[128:129, :] = b3_ref[...].reshape(1, 256)

    xt = flat_ref[...]
    h1_ref[...] = jnp.maximum(
        jax.lax.dot_general(xt, w1s_ref[0:3, :], (((0,), (0,)), ((), ())),
                            preferred_element_type=jnp.float32)
        + b1s_ref[...], 0.0)
    h2_ref[...] = jnp.maximum(
        jnp.dot(h1_ref[...], w2s_ref[...],
                preferred_element_type=jnp.float32), 0.0)
    h = jnp.maximum(
        jnp.dot(h2_ref[...], w3s_ref[...],
                preferred_element_type=jnp.float32), 0.0)

    cut = jnp.transpose(cu_ref[...].reshape(1, 17))
    lo = cut[:16]
    hi = cut[1:17]
    invc = 1.0 / jnp.maximum(hi - lo, 1).astype(jnp.float32)
    row = jax.lax.broadcasted_iota(jnp.int32, (_B, _CHUNK), 1) + i * _CHUNK
    member = (row >= lo) & (row < hi)
    onehot = member.astype(jnp.float32) * invc
    part = jnp.dot(onehot, h, preferred_element_type=jnp.float32)

    @pl.when(i == 0)
    def _():
        acc_ref[...] = part

    @pl.when(i > 0)
    def _():
        acc_ref[...] += part

    @pl.when(i == pl.num_programs(0) - 1)
    def _():
        pooled = acc_ref[...]
        z = jnp.maximum(
            jnp.dot(pooled, w4_ref[...], preferred_element_type=jnp.float32)
            + b4_ref[...], 0.0)
        o = jax.lax.dot_general(
            z, w5t_ref[...], (((1,), (1,)), ((), ())),
            preferred_element_type=jnp.float32) + b5_ref[...]
        out_ref[...] = jax.nn.sigmoid(o)


def kernel(flat, cu_seqlens, W1, b1, W2, b2, W3, b3, W4, b4, W5, b5):
    total, d_in = flat.shape
    n_chunks = total // _CHUNK

    full = lambda shape: pl.BlockSpec(shape, lambda i: tuple(0 for _ in shape))
    in_specs = [
        pl.BlockSpec((17,), lambda i: (0,)),
        pl.BlockSpec((3, _CHUNK), lambda i: (0, i)),
        full(W1.shape), full((64,)),
        full(W2.shape), full((128,)),
        full(W3.shape), full((256,)),
        full(W4.shape), full((512,)),
        full((2500, 512)), full((2500,)),
    ]
    return pl.pallas_call(
        _body,
        grid=(n_chunks,),
        in_specs=in_specs,
        out_specs=full((_B, 2500)),
        out_shape=jax.ShapeDtypeStruct((_B, 2500), jnp.float32),
        scratch_shapes=[
            pltpu.VMEM((8, 72), jnp.float32),
            pltpu.VMEM((1, 72), jnp.float32),
            pltpu.VMEM((72, 136), jnp.float32),
            pltpu.VMEM((136, 256), jnp.float32),
            pltpu.VMEM((_B, 256), jnp.float32),
            pltpu.VMEM((_CHUNK, 72), jnp.float32),
            pltpu.VMEM((_CHUNK, 136), jnp.float32),
        ],
        compiler_params=pltpu.CompilerParams(
            dimension_semantics=("arbitrary",)),
    )(cu_seqlens, flat.T, W1, b1, W2, b2, W3, b3, W4, b4, W5.T, b5)


# speedup vs baseline: 3.1109x; 1.0406x over previous
import jax
import jax.numpy as jnp
from jax.experimental import pallas as pl
from jax.experimental.pallas import tpu as pltpu

_B = 16
_CHUNK = 16384


def _body(cu_ref, flat_ref, w1_ref, b1_ref, w2_ref, b2_ref, w3_ref, b3_ref,
          w4_ref, b4_ref, w5t_ref, b5_ref, out_ref,
          w1s_ref, b1s_ref, w2s_ref, w3s_ref, acc_ref, h1_ref, h2_ref):
    i = pl.program_id(0)

    @pl.when(i == 0)
    def _():
        w1s_ref[...] = jnp.zeros_like(w1s_ref)
        w1s_ref[0:3, 0:64] = w1_ref[...]
        b1s_ref[...] = (jax.lax.broadcasted_iota(jnp.int32, (1, 72), 1)
                        == 64).astype(jnp.float32)
        b1s_ref[0:1, 0:64] = b1_ref[...].reshape(1, 64)
        w2s_ref[...] = jnp.zeros_like(w2s_ref)
        w2s_ref[0:64, 0:128] = w2_ref[...]
        w2s_ref[64:65, 0:128] = b2_ref[...].reshape(1, 128)
        w2s_ref[64:65, 128:136] = (
            jax.lax.broadcasted_iota(jnp.int32, (1, 8), 1)
            == 0).astype(jnp.float32)
        w3s_ref[...] = jnp.zeros_like(w3s_ref)
        w3s_ref[0:128, :] = w3_ref[...]
        w3s_ref[128:129, :] = b3_ref[...].reshape(1, 256)

    xt = flat_ref[...]
    h1_ref[...] = jnp.maximum(
        jax.lax.dot_general(xt, w1s_ref[0:3, :], (((0,), (0,)), ((), ())),
                            preferred_element_type=jnp.float32)
        + b1s_ref[...], 0.0)
    h2_ref[...] = jnp.maximum(
        jnp.dot(h1_ref[...], w2s_ref[...],
                preferred_element_type=jnp.float32), 0.0)
    h = jnp.maximum(
        jnp.dot(h2_ref[...], w3s_ref[...],
                preferred_element_type=jnp.float32), 0.0)

    cut = jnp.transpose(cu_ref[...].reshape(1, 17))
    lo = cut[:16]
    hi = cut[1:17]
    invc = 1.0 / jnp.maximum(hi - lo, 1).astype(jnp.float32)
    row = jax.lax.broadcasted_iota(jnp.int32, (_B, _CHUNK), 1) + i * _CHUNK
    member = (row >= lo) & (row < hi)
    onehot = member.astype(jnp.float32) * invc
    part = jnp.dot(onehot, h, preferred_element_type=jnp.float32)

    @pl.when(i == 0)
    def _():
        acc_ref[...] = part

    @pl.when(i > 0)
    def _():
        acc_ref[...] += part

    @pl.when(i == pl.num_programs(0) - 1)
    def _():
        pooled = acc_ref[...]
        z = jnp.maximum(
            jnp.dot(pooled, w4_ref[...], preferred_element_type=jnp.float32)
            + b4_ref[...], 0.0)
        o = jax.lax.dot_general(
            z, w5t_ref[...], (((1,), (1,)), ((), ())),
            preferred_element_type=jnp.float32) + b5_ref[...]
        out_ref[...] = jax.nn.sigmoid(o)


def kernel(flat, cu_seqlens, W1, b1, W2, b2, W3, b3, W4, b4, W5, b5):
    total, d_in = flat.shape
    n_chunks = total // _CHUNK

    full = lambda shape: pl.BlockSpec(shape, lambda i: tuple(0 for _ in shape))
    in_specs = [
        pl.BlockSpec((17,), lambda i: (0,)),
        pl.BlockSpec((3, _CHUNK), lambda i: (0, i)),
        full(W1.shape), full((64,)),
        full(W2.shape), full((128,)),
        full(W3.shape), full((256,)),
        full(W4.shape), full((512,)),
        full((2500, 512)), full((2500,)),
    ]
    return pl.pallas_call(
        _body,
        grid=(n_chunks,),
        in_specs=in_specs,
        out_specs=full((_B, 2500)),
        out_shape=jax.ShapeDtypeStruct((_B, 2500), jnp.float32),
        scratch_shapes=[
            pltpu.VMEM((8, 72), jnp.float32),
            pltpu.VMEM((1, 72), jnp.float32),
            pltpu.VMEM((72, 136), jnp.float32),
            pltpu.VMEM((136, 256), jnp.float32),
            pltpu.VMEM((_B, 256), jnp.float32),
            pltpu.VMEM((_CHUNK, 72), jnp.float32),
            pltpu.VMEM((_CHUNK, 136), jnp.float32),
        ],
        compiler_params=pltpu.CompilerParams(
            dimension_semantics=("arbitrary",)),
    )(cu_seqlens, flat.T, W1, b1, W2, b2, W3, b3, W4, b4, W5.T, b5)
